# baseline (speedup 1.0000x reference)
import functools

import jax
import jax.numpy as jnp
from jax import lax
from jax.experimental import pallas as pl
from jax.experimental.pallas import tpu as pltpu
from jax.experimental.pallas import tpu_sc as plsc

_Q, _K, _D = 1024, 100000, 128
_TOPK = 200
_THRESH = 0.05
_TK = 2048
_KP = 100352
_CH = 128
_NC = _KP // _CH
_NW = 32
_QPW = _Q // _NW
_NCAP = 256
_ECAP = 512
_NEG = -3.0


def _sim_body(q_ref, k_ref, sim_ref, m_ref):
    i = pl.program_id(0)
    qn = q_ref[...]
    kn = k_ref[...]
    sim = lax.dot_general(qn, kn, (((1,), (1,)), ((), ())),
                          preferred_element_type=jnp.float32)
    col = i * _TK + lax.broadcasted_iota(jnp.int32, (_Q, _TK), 1)
    sim = jnp.where(col < _K, sim, _NEG)
    sim_ref[...] = sim
    m_ref[0] = jnp.max(sim.reshape(_Q, _TK // _CH, _CH), axis=-1)


def _thresh_body(m_ref, t_ref):
    m = m_ref[...]

    def it(_, carry):
        lo, hi = carry
        mid = 0.5 * (lo + hi)
        cnt = jnp.sum((m > mid).astype(jnp.int32), axis=-1, keepdims=True)
        ok = cnt >= _TOPK
        return jnp.where(ok, mid, lo), jnp.where(ok, hi, mid)

    lo0 = jnp.full((_Q, 1), -1.1, jnp.float32)
    hi0 = jnp.full((_Q, 1), 1.1, jnp.float32)
    lo, _ = lax.fori_loop(0, 30, it, (lo0, hi0))
    t_ref[...] = jnp.broadcast_to(lo, (_Q, 128))


def _permute(vec16, perm16):
    dn = lax.GatherDimensionNumbers(
        offset_dims=(), collapsed_slice_dims=(0,), start_index_map=(0,))
    return lax.gather(vec16, perm16, dn, (1,),
                      mode=lax.GatherScatterMode.PROMISE_IN_BOUNDS)


def _splat(vec16, j):
    idx = jnp.full((16, 1), j, jnp.int32)
    dn = lax.GatherDimensionNumbers(
        offset_dims=(), collapsed_slice_dims=(0,), start_index_map=(0,))
    return lax.gather(vec16, idx, dn, (1,),
                      mode=lax.GatherScatterMode.PROMISE_IN_BOUNDS)


def _sc_body(sim_ref, m_ref, t_ref, val_out, idx_out,
             m_v, cid_v, idxs_v, rows_v, cand_v, candi_v, t_v, sem):
    cid_core = lax.axis_index("c")
    sid = lax.axis_index("s")
    wid = sid * 2 + cid_core
    qbase = wid * _QPW
    iota16 = lax.iota(jnp.int32, 16)

    def initc(i, _):
        cid_v[pl.ds(i * 16, 16)] = i * 16 + iota16
        return 0
    lax.fori_loop(0, (_NCAP + 16) // 16, initc, 0)

    def per_query(j, _):
        q = qbase + j
        pltpu.sync_copy(m_ref.at[q], m_v)
        pltpu.sync_copy(t_ref.at[q, pl.ds(0, 16)], t_v)
        tq = t_v[...]
        qrow = q * _NC

        def mscan(g, cnt):
            m16 = m_v[pl.ds(g * 16, 16)]
            msk = m16 > tq
            ids = qrow + g * 16 + iota16
            mi = jnp.where(msk, 1, 0)
            pos = cnt + plsc.cumsum(mi) - 1
            plsc.store_scatter(cid_v, [pos], ids, mask=msk)
            return jnp.minimum(cnt + jnp.sum(mi), _NCAP)
        nch = lax.fori_loop(0, _NC // 16, mscan, 0)

        def cpix(i, _):
            idxs_v[0, pl.ds(i * 16, 16)] = cid_v[pl.ds(i * 16, 16)]
            idxs_v[1, pl.ds(i * 16, 16)] = cid_v[pl.ds(128 + i * 16, 16)]
            return 0
        lax.fori_loop(0, 8, cpix, 0)

        cp0 = pltpu.async_copy(sim_ref.at[idxs_v.at[0]],
                               rows_v.at[pl.ds(0, 128)], sem)
        cp1 = pltpu.async_copy(sim_ref.at[idxs_v.at[1]],
                               rows_v.at[pl.ds(128, 128)], sem)
        cp0.wait()
        cp1.wait()

        negv = jnp.full((16,), _NEG, jnp.float32)
        def clr(i, _):
            cand_v[pl.ds(i * 16, 16)] = negv
            return 0
        lax.fori_loop(0, (_ECAP + 32) // 16, clr, 0)

        qrow_v = jnp.full((16,), qrow, jnp.int32)
        nch_v = jnp.full((16,), nch, jnp.int32)

        def fscan(g, cnt):
            cid16 = cid_v[pl.ds(g * 16, 16)]
            for j2 in range(16):
                cvec = _splat(cid16, j2)
                kbase = (cvec - qrow_v) * _CH
                valid = jnp.full((16,), g * 16 + j2, jnp.int32) < nch_v
                for v in range(_CH // 16):
                    data = rows_v[g * 16 + j2, pl.ds(v * 16, 16)]
                    msk = jnp.logical_and(data > tq, valid)
                    kidx = kbase + (v * 16) + iota16
                    mi = jnp.where(msk, 1, 0)
                    pos = cnt + plsc.cumsum(mi) - 1
                    plsc.store_scatter(cand_v, [pos], data, mask=msk)
                    plsc.store_scatter(candi_v, [pos], kidx, mask=msk)
                    cnt = jnp.minimum(cnt + jnp.sum(mi), _ECAP)
            return cnt
        lax.fori_loop(0, _NCAP // 16, fscan, 0)

        for r in range(_ECAP // 16):
            kk = cand_v[pl.ds(r * 16, 16)]
            vv = candi_v[pl.ds(r * 16, 16)]
            sk, sv = plsc.sort_key_val(kk, vv, descending=(r % 2 == 0))
            cand_v[pl.ds(r * 16, 16)] = sk
            candi_v[pl.ds(r * 16, 16)] = sv

        nreg = _ECAP // 16
        for kv in (2, 4, 8, 16, 32):
            jv = kv // 2
            while jv >= 1:
                for r in range(nreg):
                    if r & jv:
                        continue
                    asc = (r & kv) != 0
                    ak = cand_v[pl.ds(r * 16, 16)]
                    bk = cand_v[pl.ds((r + jv) * 16, 16)]
                    av = candi_v[pl.ds(r * 16, 16)]
                    bv = candi_v[pl.ds((r + jv) * 16, 16)]
                    swap = (ak > bk) if asc else (ak < bk)
                    cand_v[pl.ds(r * 16, 16)] = jnp.where(swap, bk, ak)
                    cand_v[pl.ds((r + jv) * 16, 16)] = jnp.where(swap, ak, bk)
                    candi_v[pl.ds(r * 16, 16)] = jnp.where(swap, bv, av)
                    candi_v[pl.ds((r + jv) * 16, 16)] = jnp.where(swap, av, bv)
                jv //= 2
            for r in range(nreg):
                kk = cand_v[pl.ds(r * 16, 16)]
                vv = candi_v[pl.ds(r * 16, 16)]
                sk, sv = plsc.sort_key_val(kk, vv,
                                           descending=(r & kv) == 0)
                cand_v[pl.ds(r * 16, 16)] = sk
                candi_v[pl.ds(r * 16, 16)] = sv

        perm = (iota16 ^ 1).reshape(16, 1)
        evenlane = (iota16 & 1) == 0
        for off in (0, 1, 0):
            def tbfix(r, _, off=off):
                base = r * 16 + off
                vv = cand_v[pl.ds(base, 16)]
                ii = candi_v[pl.ds(base, 16)]
                pv = _permute(vv, perm)
                pi = _permute(ii, perm)
                eq = vv == pv
                mn = jnp.minimum(ii, pi)
                mx = jnp.maximum(ii, pi)
                candi_v[pl.ds(base, 16)] = jnp.where(
                    eq, jnp.where(evenlane, mn, mx), ii)
                return 0
            lax.fori_loop(0, _ECAP // 16, tbfix, 0)

        pltpu.sync_copy(cand_v.at[pl.ds(0, 256)], val_out.at[q])
        pltpu.sync_copy(candi_v.at[pl.ds(0, 256)], idx_out.at[q])
        return 0

    lax.fori_loop(0, _QPW, per_query, 0)


_sc_select = functools.partial(
    pl.kernel,
    out_type=[jax.ShapeDtypeStruct((_Q, 256), jnp.float32),
              jax.ShapeDtypeStruct((_Q, 256), jnp.int32)],
    mesh=plsc.VectorSubcoreMesh(core_axis_name="c", subcore_axis_name="s"),
    compiler_params=pltpu.CompilerParams(needs_layout_passes=False),
    scratch_types=[
        pltpu.VMEM((_NC,), jnp.float32),
        pltpu.VMEM((_NCAP + 16,), jnp.int32),
        pltpu.VMEM((2, 128), jnp.int32),
        pltpu.VMEM((_NCAP, _CH), jnp.float32),
        pltpu.VMEM((_ECAP + 32,), jnp.float32),
        pltpu.VMEM((_ECAP + 32,), jnp.int32),
        pltpu.VMEM((16,), jnp.float32),
        pltpu.SemaphoreType.DMA,
    ],
)(_sc_body)


def kernel(queries, keys):
    qn = queries / jnp.sqrt(jnp.sum(queries ** 2, axis=-1, keepdims=True) + 1e-8)
    kn = keys / jnp.sqrt(jnp.sum(keys ** 2, axis=-1, keepdims=True) + 1e-8)
    kp = jnp.pad(kn, ((0, _KP - _K), (0, 0)))

    sim, m = pl.pallas_call(
        _sim_body,
        grid=(_KP // _TK,),
        in_specs=[
            pl.BlockSpec((_Q, _D), lambda i: (0, 0)),
            pl.BlockSpec((_TK, _D), lambda i: (i, 0)),
        ],
        out_specs=[
            pl.BlockSpec((_Q, _TK), lambda i: (0, i)),
            pl.BlockSpec((1, _Q, _TK // _CH), lambda i: (i, 0, 0)),
        ],
        out_shape=[
            jax.ShapeDtypeStruct((_Q, _KP), jnp.float32),
            jax.ShapeDtypeStruct((_KP // _TK, _Q, _TK // _CH), jnp.float32),
        ],
    )(qn, kp)
    m = m.transpose(1, 0, 2).reshape(_Q, _NC)

    t = pl.pallas_call(
        _thresh_body,
        out_shape=jax.ShapeDtypeStruct((_Q, 128), jnp.float32),
    )(m)

    sim2 = sim.reshape(_Q * _NC, _CH)
    val, idx = _sc_select(sim2, m, t)

    score = val[:, :_TOPK]
    end = idx[:, :_TOPK]
    matched = score > _THRESH
    matched_vocab = jnp.where(matched, end, -1)
    value = jnp.where(matched, score, 0.0)
    return score, end, matched, matched_vocab, value


# speedup vs baseline: 1.2287x; 1.2287x over previous
import functools

import jax
import jax.numpy as jnp
from jax import lax
from jax.experimental import pallas as pl
from jax.experimental.pallas import tpu as pltpu
from jax.experimental.pallas import tpu_sc as plsc

_Q, _K, _D = 1024, 100000, 128
_TOPK = 200
_THRESH = 0.05
_TK = 2048
_KP = 100352
_CH = 128
_NC = _KP // _CH
_NW = 32
_QPW = _Q // _NW
_NCAP = 256
_ECAP = 512
_NEG = -3.0


def _sim_body(q_ref, k_ref, sim_ref, m_ref):
    i = pl.program_id(0)
    qn = q_ref[...]
    kn = k_ref[...]
    sim = lax.dot_general(qn, kn, (((1,), (1,)), ((), ())),
                          preferred_element_type=jnp.float32)
    col = i * _TK + lax.broadcasted_iota(jnp.int32, (_Q, _TK), 1)
    sim = jnp.where(col < _K, sim, _NEG)
    sim_ref[...] = sim
    m_ref[0] = jnp.max(sim.reshape(_Q, _TK // _CH, _CH), axis=-1)


def _thresh_body(m_ref, t_ref):
    m = m_ref[...]

    def it(_, carry):
        lo, hi = carry
        mid = 0.5 * (lo + hi)
        cnt = jnp.sum((m > mid).astype(jnp.int32), axis=-1, keepdims=True)
        ok = cnt >= _TOPK
        return jnp.where(ok, mid, lo), jnp.where(ok, hi, mid)

    lo0 = jnp.full((_Q, 1), -1.1, jnp.float32)
    hi0 = jnp.full((_Q, 1), 1.1, jnp.float32)
    lo, _ = lax.fori_loop(0, 30, it, (lo0, hi0))
    t_ref[...] = jnp.broadcast_to(lo, (_Q, 16))


def _permute(vec16, perm16):
    dn = lax.GatherDimensionNumbers(
        offset_dims=(), collapsed_slice_dims=(0,), start_index_map=(0,))
    return lax.gather(vec16, perm16, dn, (1,),
                      mode=lax.GatherScatterMode.PROMISE_IN_BOUNDS)


def _splat(vec16, j):
    idx = jnp.full((16, 1), j, jnp.int32)
    dn = lax.GatherDimensionNumbers(
        offset_dims=(), collapsed_slice_dims=(0,), start_index_map=(0,))
    return lax.gather(vec16, idx, dn, (1,),
                      mode=lax.GatherScatterMode.PROMISE_IN_BOUNDS)


def _sc_body(sim_ref, m_ref, t_ref, val_out, idx_out,
             m_all_v, t_all_v, cid_v, idxs_v, rows_v, cand_v, candi_v, sem):
    cid_core = lax.axis_index("c")
    sid = lax.axis_index("s")
    wid = sid * 2 + cid_core
    qbase = wid * _QPW
    iota16 = lax.iota(jnp.int32, 16)
    zero16 = jnp.zeros((16,), jnp.int32)

    pltpu.sync_copy(m_ref.at[pl.ds(qbase, _QPW)], m_all_v)
    pltpu.sync_copy(t_ref.at[pl.ds(qbase, _QPW)], t_all_v)

    def initc(i, _):
        cid_v[pl.ds(i * 16, 16)] = i * 16 + iota16
        return 0
    lax.fori_loop(0, (_NCAP + 16) // 16, initc, 0)

    def per_query(j, _):
        q = qbase + j
        tq = t_all_v[j, pl.ds(0, 16)]
        qrow = q * _NC

        def mscan(g, cnt_vec):
            m16 = m_all_v[j, pl.ds(g * 16, 16)]
            msk = m16 > tq
            mi = jnp.where(msk, 1, 0)
            pos = cnt_vec + plsc.cumsum(mi) - 1
            ids = qrow + g * 16 + iota16
            plsc.store_scatter(cid_v, [pos], ids, mask=msk)
            pc = plsc.all_reduce_population_count(msk)
            return jnp.minimum(cnt_vec + pc, _NCAP)
        nch_vec = lax.fori_loop(0, _NC // 16, mscan, zero16)
        nch = jnp.max(nch_vec)

        def cpix(i, _):
            idxs_v[0, pl.ds(i * 16, 16)] = cid_v[pl.ds(i * 16, 16)]
            idxs_v[1, pl.ds(i * 16, 16)] = cid_v[pl.ds(128 + i * 16, 16)]
            return 0
        lax.fori_loop(0, 8, cpix, 0)

        cp0 = pltpu.async_copy(sim_ref.at[idxs_v.at[0]],
                               rows_v.at[pl.ds(0, 128)], sem)
        cp1 = pltpu.async_copy(sim_ref.at[idxs_v.at[1]],
                               rows_v.at[pl.ds(128, 128)], sem)

        negv = jnp.full((16,), _NEG, jnp.float32)
        def clr(i, _):
            cand_v[pl.ds(i * 16, 16)] = negv
            return 0
        lax.fori_loop(0, (_ECAP + 32) // 16, clr, 0)

        cp0.wait()
        cp1.wait()

        qrow_v = jnp.full((16,), qrow, jnp.int32)
        nch_v = nch_vec

        def fscan(g, cnt_vec):
            cid16 = cid_v[pl.ds(g * 16, 16)]
            for j2 in range(16):
                cvec = _splat(cid16, j2)
                kbase = (cvec - qrow_v) * _CH
                valid = jnp.full((16,), g * 16 + j2, jnp.int32) < nch_v
                for v in range(_CH // 16):
                    data = rows_v[g * 16 + j2, pl.ds(v * 16, 16)]
                    msk = jnp.logical_and(data > tq, valid)
                    mi = jnp.where(msk, 1, 0)
                    pos = cnt_vec + plsc.cumsum(mi) - 1
                    kidx = kbase + (v * 16) + iota16
                    plsc.store_scatter(cand_v, [pos], data, mask=msk)
                    plsc.store_scatter(candi_v, [pos], kidx, mask=msk)
                    pc = plsc.all_reduce_population_count(msk)
                    cnt_vec = jnp.minimum(cnt_vec + pc, _ECAP)
            return cnt_vec
        ngroups = (nch + 15) // 16
        lax.fori_loop(0, ngroups, fscan, zero16)

        for r in range(_ECAP // 16):
            kk = cand_v[pl.ds(r * 16, 16)]
            vv = candi_v[pl.ds(r * 16, 16)]
            sk, sv = plsc.sort_key_val(kk, vv, descending=(r % 2 == 0))
            cand_v[pl.ds(r * 16, 16)] = sk
            candi_v[pl.ds(r * 16, 16)] = sv

        nreg = _ECAP // 16
        for kv in (2, 4, 8, 16, 32):
            jv = kv // 2
            while jv >= 1:
                for r in range(nreg):
                    if r & jv:
                        continue
                    asc = (r & kv) != 0
                    ak = cand_v[pl.ds(r * 16, 16)]
                    bk = cand_v[pl.ds((r + jv) * 16, 16)]
                    av = candi_v[pl.ds(r * 16, 16)]
                    bv = candi_v[pl.ds((r + jv) * 16, 16)]
                    swap = (ak > bk) if asc else (ak < bk)
                    cand_v[pl.ds(r * 16, 16)] = jnp.where(swap, bk, ak)
                    cand_v[pl.ds((r + jv) * 16, 16)] = jnp.where(swap, ak, bk)
                    candi_v[pl.ds(r * 16, 16)] = jnp.where(swap, bv, av)
                    candi_v[pl.ds((r + jv) * 16, 16)] = jnp.where(swap, av, bv)
                jv //= 2
            for r in range(nreg):
                kk = cand_v[pl.ds(r * 16, 16)]
                vv = candi_v[pl.ds(r * 16, 16)]
                sk, sv = plsc.sort_key_val(kk, vv,
                                           descending=(r & kv) == 0)
                cand_v[pl.ds(r * 16, 16)] = sk
                candi_v[pl.ds(r * 16, 16)] = sv

        perm = (iota16 ^ 1).reshape(16, 1)
        evenlane = (iota16 & 1) == 0
        for off in (0, 1, 0):
            for r in range(13):
                base = r * 16 + off
                vv = cand_v[pl.ds(base, 16)]
                ii = candi_v[pl.ds(base, 16)]
                pv = _permute(vv, perm)
                pi = _permute(ii, perm)
                eq = vv == pv
                mn = jnp.minimum(ii, pi)
                mx = jnp.maximum(ii, pi)
                candi_v[pl.ds(base, 16)] = jnp.where(
                    eq, jnp.where(evenlane, mn, mx), ii)

        pltpu.sync_copy(cand_v.at[pl.ds(0, 256)], val_out.at[q])
        pltpu.sync_copy(candi_v.at[pl.ds(0, 256)], idx_out.at[q])
        return 0

    lax.fori_loop(0, _QPW, per_query, 0)


_sc_select = functools.partial(
    pl.kernel,
    out_type=[jax.ShapeDtypeStruct((_Q, 256), jnp.float32),
              jax.ShapeDtypeStruct((_Q, 256), jnp.int32)],
    mesh=plsc.VectorSubcoreMesh(core_axis_name="c", subcore_axis_name="s"),
    compiler_params=pltpu.CompilerParams(needs_layout_passes=False),
    scratch_types=[
        pltpu.VMEM((_QPW, _NC), jnp.float32),
        pltpu.VMEM((_QPW, 16), jnp.float32),
        pltpu.VMEM((_NCAP + 16,), jnp.int32),
        pltpu.VMEM((2, 128), jnp.int32),
        pltpu.VMEM((_NCAP, _CH), jnp.float32),
        pltpu.VMEM((_ECAP + 32,), jnp.float32),
        pltpu.VMEM((_ECAP + 32,), jnp.int32),
        pltpu.SemaphoreType.DMA,
    ],
)(_sc_body)


def kernel(queries, keys):
    qn = queries / jnp.sqrt(jnp.sum(queries ** 2, axis=-1, keepdims=True) + 1e-8)
    kn = keys / jnp.sqrt(jnp.sum(keys ** 2, axis=-1, keepdims=True) + 1e-8)
    kp = jnp.pad(kn, ((0, _KP - _K), (0, 0)))

    sim, m = pl.pallas_call(
        _sim_body,
        grid=(_KP // _TK,),
        in_specs=[
            pl.BlockSpec((_Q, _D), lambda i: (0, 0)),
            pl.BlockSpec((_TK, _D), lambda i: (i, 0)),
        ],
        out_specs=[
            pl.BlockSpec((_Q, _TK), lambda i: (0, i)),
            pl.BlockSpec((1, _Q, _TK // _CH), lambda i: (i, 0, 0)),
        ],
        out_shape=[
            jax.ShapeDtypeStruct((_Q, _KP), jnp.float32),
            jax.ShapeDtypeStruct((_KP // _TK, _Q, _TK // _CH), jnp.float32),
        ],
    )(qn, kp)
    m = m.transpose(1, 0, 2).reshape(_Q, _NC)

    t = pl.pallas_call(
        _thresh_body,
        out_shape=jax.ShapeDtypeStruct((_Q, 16), jnp.float32),
    )(m)

    sim2 = sim.reshape(_Q * _NC, _CH)
    val, idx = _sc_select(sim2, m, t)

    score = val[:, :_TOPK]
    end = idx[:, :_TOPK]
    matched = score > _THRESH
    matched_vocab = jnp.where(matched, end, -1)
    value = jnp.where(matched, score, 0.0)
    return score, end, matched, matched_vocab, value


# speedup vs baseline: 1.4331x; 1.1664x over previous
import functools

import jax
import jax.numpy as jnp
from jax import lax
from jax.experimental import pallas as pl
from jax.experimental.pallas import tpu as pltpu
from jax.experimental.pallas import tpu_sc as plsc

_Q, _K, _D = 1024, 100000, 128
_TOPK = 200
_THRESH = 0.05
_TK = 2048
_KP = 100352
_CH = 128
_NC = _KP // _CH
_NW = 32
_QPW = _Q // _NW
_NCAP = 256
_ECAP = 512
_NEG = -3.0


def _sim_body(q_ref, k_ref, sim_ref, m_ref):
    i = pl.program_id(0)
    qn = q_ref[...]
    kn = k_ref[...]
    sim = lax.dot_general(qn, kn, (((1,), (1,)), ((), ())),
                          preferred_element_type=jnp.float32)
    col = i * _TK + lax.broadcasted_iota(jnp.int32, (_Q, _TK), 1)
    sim = jnp.where(col < _K, sim, _NEG)
    sim3 = sim.reshape(_Q, _TK // _CH, _CH)
    sim_ref[...] = sim3
    m_ref[0] = jnp.max(sim3, axis=-1)


def _thresh_body(m_ref, t_ref):
    m = m_ref[...]

    def it(_, carry):
        lo, hi = carry
        mid = 0.5 * (lo + hi)
        cnt = jnp.sum((m > mid).astype(jnp.int32), axis=-1, keepdims=True)
        ok = cnt >= _TOPK
        return jnp.where(ok, mid, lo), jnp.where(ok, hi, mid)

    lo0 = jnp.full((_Q, 1), -1.1, jnp.float32)
    hi0 = jnp.full((_Q, 1), 1.1, jnp.float32)
    lo, _ = lax.fori_loop(0, 30, it, (lo0, hi0))
    t_ref[...] = jnp.broadcast_to(lo, (_Q, 16))


def _permute(vec16, perm16):
    dn = lax.GatherDimensionNumbers(
        offset_dims=(), collapsed_slice_dims=(0,), start_index_map=(0,))
    return lax.gather(vec16, perm16, dn, (1,),
                      mode=lax.GatherScatterMode.PROMISE_IN_BOUNDS)


def _splat(vec16, j):
    idx = jnp.full((16, 1), j, jnp.int32)
    dn = lax.GatherDimensionNumbers(
        offset_dims=(), collapsed_slice_dims=(0,), start_index_map=(0,))
    return lax.gather(vec16, idx, dn, (1,),
                      mode=lax.GatherScatterMode.PROMISE_IN_BOUNDS)


def _sc_body(sim_ref, m_ref, t_ref, val_out, idx_out,
             m_all_v, t_all_v, cid_v, idxs_v, rows_v, cand_v, candi_v, sem):
    cid_core = lax.axis_index("c")
    sid = lax.axis_index("s")
    wid = sid * 2 + cid_core
    qbase = wid * _QPW
    iota16 = lax.iota(jnp.int32, 16)
    zero16 = jnp.zeros((16,), jnp.int32)

    pltpu.sync_copy(m_ref.at[pl.ds(qbase, _QPW)], m_all_v)
    pltpu.sync_copy(t_ref.at[pl.ds(qbase, _QPW)], t_all_v)

    def initc(i, _):
        cid_v[pl.ds(i * 16, 16)] = i * 16 + iota16
        return 0
    lax.fori_loop(0, (_NCAP + 16) // 16, initc, 0)

    def per_query(j, _):
        q = qbase + j
        tq = t_all_v[j, pl.ds(0, 16)]
        qrow = q * _NC

        def mscan(g, cnt_vec):
            m16 = m_all_v[j, pl.ds(g * 16, 16)]
            msk = m16 > tq
            mi = jnp.where(msk, 1, 0)
            pos = cnt_vec + plsc.cumsum(mi) - 1
            ids = qrow + g * 16 + iota16
            plsc.store_scatter(cid_v, [pos], ids, mask=msk)
            pc = plsc.all_reduce_population_count(msk)
            return jnp.minimum(cnt_vec + pc, _NCAP)
        nch_vec = lax.fori_loop(0, _NC // 16, mscan, zero16)
        nch = jnp.max(nch_vec)

        def cpix(i, _):
            idxs_v[0, pl.ds(i * 16, 16)] = cid_v[pl.ds(i * 16, 16)]
            idxs_v[1, pl.ds(i * 16, 16)] = cid_v[pl.ds(128 + i * 16, 16)]
            return 0
        lax.fori_loop(0, 8, cpix, 0)

        cp0 = pltpu.async_copy(sim_ref.at[idxs_v.at[0]],
                               rows_v.at[pl.ds(0, 128)], sem)
        cp1 = pltpu.async_copy(sim_ref.at[idxs_v.at[1]],
                               rows_v.at[pl.ds(128, 128)], sem)

        negv = jnp.full((16,), _NEG, jnp.float32)
        def clr(i, _):
            cand_v[pl.ds(i * 16, 16)] = negv
            return 0
        lax.fori_loop(0, (_ECAP + 32) // 16, clr, 0)

        cp0.wait()
        cp1.wait()

        qrow_v = jnp.full((16,), qrow, jnp.int32)
        nch_v = nch_vec

        def fscan(g, cnt_vec):
            cid16 = cid_v[pl.ds(g * 16, 16)]
            for j2 in range(16):
                cvec = _splat(cid16, j2)
                kbase = (cvec - qrow_v) * _CH
                valid = jnp.full((16,), g * 16 + j2, jnp.int32) < nch_v
                for v in range(_CH // 16):
                    data = rows_v[g * 16 + j2, pl.ds(v * 16, 16)]
                    msk = jnp.logical_and(data > tq, valid)
                    mi = jnp.where(msk, 1, 0)
                    pos = cnt_vec + plsc.cumsum(mi) - 1
                    kidx = kbase + (v * 16) + iota16
                    plsc.store_scatter(cand_v, [pos], data, mask=msk)
                    plsc.store_scatter(candi_v, [pos], kidx, mask=msk)
                    pc = plsc.all_reduce_population_count(msk)
                    cnt_vec = jnp.minimum(cnt_vec + pc, _ECAP)
            return cnt_vec
        ngroups = (nch + 15) // 16
        lax.fori_loop(0, ngroups, fscan, zero16)

        for r in range(_ECAP // 16):
            kk = cand_v[pl.ds(r * 16, 16)]
            vv = candi_v[pl.ds(r * 16, 16)]
            sk, sv = plsc.sort_key_val(kk, vv, descending=(r % 2 == 0))
            cand_v[pl.ds(r * 16, 16)] = sk
            candi_v[pl.ds(r * 16, 16)] = sv

        nreg = _ECAP // 16
        for kv in (2, 4, 8, 16, 32):
            jv = kv // 2
            while jv >= 1:
                for r in range(nreg):
                    if r & jv:
                        continue
                    asc = (r & kv) != 0
                    ak = cand_v[pl.ds(r * 16, 16)]
                    bk = cand_v[pl.ds((r + jv) * 16, 16)]
                    av = candi_v[pl.ds(r * 16, 16)]
                    bv = candi_v[pl.ds((r + jv) * 16, 16)]
                    swap = (ak > bk) if asc else (ak < bk)
                    cand_v[pl.ds(r * 16, 16)] = jnp.where(swap, bk, ak)
                    cand_v[pl.ds((r + jv) * 16, 16)] = jnp.where(swap, ak, bk)
                    candi_v[pl.ds(r * 16, 16)] = jnp.where(swap, bv, av)
                    candi_v[pl.ds((r + jv) * 16, 16)] = jnp.where(swap, av, bv)
                jv //= 2
            for r in range(nreg):
                kk = cand_v[pl.ds(r * 16, 16)]
                vv = candi_v[pl.ds(r * 16, 16)]
                sk, sv = plsc.sort_key_val(kk, vv,
                                           descending=(r & kv) == 0)
                cand_v[pl.ds(r * 16, 16)] = sk
                candi_v[pl.ds(r * 16, 16)] = sv

        perm = (iota16 ^ 1).reshape(16, 1)
        evenlane = (iota16 & 1) == 0
        for off in (0, 1, 0):
            for r in range(13):
                base = r * 16 + off
                vv = cand_v[pl.ds(base, 16)]
                ii = candi_v[pl.ds(base, 16)]
                pv = _permute(vv, perm)
                pi = _permute(ii, perm)
                eq = vv == pv
                mn = jnp.minimum(ii, pi)
                mx = jnp.maximum(ii, pi)
                candi_v[pl.ds(base, 16)] = jnp.where(
                    eq, jnp.where(evenlane, mn, mx), ii)

        pltpu.sync_copy(cand_v.at[pl.ds(0, 256)], val_out.at[q])
        pltpu.sync_copy(candi_v.at[pl.ds(0, 256)], idx_out.at[q])
        return 0

    lax.fori_loop(0, _QPW, per_query, 0)


_sc_select = functools.partial(
    pl.kernel,
    out_type=[jax.ShapeDtypeStruct((_Q, 256), jnp.float32),
              jax.ShapeDtypeStruct((_Q, 256), jnp.int32)],
    mesh=plsc.VectorSubcoreMesh(core_axis_name="c", subcore_axis_name="s"),
    compiler_params=pltpu.CompilerParams(needs_layout_passes=False),
    scratch_types=[
        pltpu.VMEM((_QPW, _NC), jnp.float32),
        pltpu.VMEM((_QPW, 16), jnp.float32),
        pltpu.VMEM((_NCAP + 16,), jnp.int32),
        pltpu.VMEM((2, 128), jnp.int32),
        pltpu.VMEM((_NCAP, _CH), jnp.float32),
        pltpu.VMEM((_ECAP + 32,), jnp.float32),
        pltpu.VMEM((_ECAP + 32,), jnp.int32),
        pltpu.SemaphoreType.DMA,
    ],
)(_sc_body)


def kernel(queries, keys):
    qn = queries / jnp.sqrt(jnp.sum(queries ** 2, axis=-1, keepdims=True) + 1e-8)
    kn = keys / jnp.sqrt(jnp.sum(keys ** 2, axis=-1, keepdims=True) + 1e-8)

    sim, m = pl.pallas_call(
        _sim_body,
        grid=(_KP // _TK,),
        in_specs=[
            pl.BlockSpec((_Q, _D), lambda i: (0, 0)),
            pl.BlockSpec((_TK, _D), lambda i: (i, 0)),
        ],
        out_specs=[
            pl.BlockSpec((_Q, _TK // _CH, _CH), lambda i: (0, i, 0)),
            pl.BlockSpec((1, _Q, _TK // _CH), lambda i: (i, 0, 0)),
        ],
        out_shape=[
            jax.ShapeDtypeStruct((_Q, _NC, _CH), jnp.float32),
            jax.ShapeDtypeStruct((_KP // _TK, _Q, _TK // _CH), jnp.float32),
        ],
    )(qn, kn)
    m = m.transpose(1, 0, 2).reshape(_Q, _NC)

    t = pl.pallas_call(
        _thresh_body,
        out_shape=jax.ShapeDtypeStruct((_Q, 16), jnp.float32),
    )(m)

    sim2 = sim.reshape(_Q * _NC, _CH)
    val, idx = _sc_select(sim2, m, t)

    score = val[:, :_TOPK]
    end = idx[:, :_TOPK]
    matched = score > _THRESH
    matched_vocab = jnp.where(matched, end, -1)
    value = jnp.where(matched, score, 0.0)
    return score, end, matched, matched_vocab, value


# speedup vs baseline: 3.9019x; 2.7228x over previous
import functools

import jax
import jax.numpy as jnp
from jax import lax
from jax.experimental import pallas as pl
from jax.experimental.pallas import tpu as pltpu
from jax.experimental.pallas import tpu_sc as plsc

_Q, _K, _D = 1024, 100000, 128
_TOPK = 200
_THRESH = 0.05
_TK = 2048
_KP = 100352
_CH = 128
_NC = _KP // _CH
_NW = 32
_QPW = _Q // _NW
_NCAP = 256
_ECAP = 512
_NEG = -3.0


def _sim_body(q_ref, k_ref, sim_ref, m_ref):
    i = pl.program_id(0)
    qn = q_ref[...]
    kn = k_ref[...]
    sim = lax.dot_general(qn, kn, (((1,), (1,)), ((), ())),
                          preferred_element_type=jnp.float32)
    col = i * _TK + lax.broadcasted_iota(jnp.int32, (_Q, _TK), 1)
    sim = jnp.where(col < _K, sim, _NEG)
    sim3 = sim.reshape(_Q, _TK // _CH, _CH)
    sim_ref[...] = sim3
    m_ref[0] = jnp.max(sim3, axis=-1)


def _thresh_body(m_ref, t_ref):
    m = m_ref[...]

    def it(_, carry):
        lo, hi = carry
        mid = 0.5 * (lo + hi)
        cnt = jnp.sum((m > mid).astype(jnp.int32), axis=-1, keepdims=True)
        ok = cnt >= _TOPK
        return jnp.where(ok, mid, lo), jnp.where(ok, hi, mid)

    lo0 = jnp.full((_Q, 1), -1.1, jnp.float32)
    hi0 = jnp.full((_Q, 1), 1.1, jnp.float32)
    lo, _ = lax.fori_loop(0, 30, it, (lo0, hi0))
    t_ref[...] = jnp.broadcast_to(lo, (_Q, 16))


def _permute(vec16, perm16):
    dn = lax.GatherDimensionNumbers(
        offset_dims=(), collapsed_slice_dims=(0,), start_index_map=(0,))
    return lax.gather(vec16, perm16, dn, (1,),
                      mode=lax.GatherScatterMode.PROMISE_IN_BOUNDS)


def _splat(vec16, j):
    idx = jnp.full((16, 1), j, jnp.int32)
    dn = lax.GatherDimensionNumbers(
        offset_dims=(), collapsed_slice_dims=(0,), start_index_map=(0,))
    return lax.gather(vec16, idx, dn, (1,),
                      mode=lax.GatherScatterMode.PROMISE_IN_BOUNDS)


def _sc_body(sim_ref, m_ref, t_ref, val_out, idx_out,
             m_all_v, t_all_v, cid_v, idxs_v, rows_v, cand_v, candi_v, sem):
    cid_core = lax.axis_index("c")
    sid = lax.axis_index("s")
    wid = sid * 2 + cid_core
    qbase = wid * _QPW
    iota16 = lax.iota(jnp.int32, 16)
    zero16 = jnp.zeros((16,), jnp.int32)

    pltpu.sync_copy(m_ref.at[pl.ds(qbase, _QPW)], m_all_v)
    pltpu.sync_copy(t_ref.at[pl.ds(qbase, _QPW)], t_all_v)

    def initc(i, _):
        cid_v[pl.ds(i * 16, 16)] = i * 16 + iota16
        return 0
    lax.fori_loop(0, (_NCAP + 16) // 16, initc, 0)

    def per_query(j, _):
        q = qbase + j
        tq = t_all_v[j, pl.ds(0, 16)]
        qrow = q * _NC

        def mscan(g, cnt_vec):
            m16 = m_all_v[j, pl.ds(g * 16, 16)]
            msk = m16 > tq
            mi = jnp.where(msk, 1, 0)
            pos = cnt_vec + plsc.cumsum(mi) - 1
            ids = qrow + g * 16 + iota16
            plsc.store_scatter(cid_v, [pos], ids, mask=msk)
            pc = plsc.all_reduce_population_count(msk)
            return jnp.minimum(cnt_vec + pc, _NCAP)
        nch_vec = lax.fori_loop(0, _NC // 16, mscan, zero16)
        nch = jnp.max(nch_vec)

        def cpix(i, _):
            idxs_v[0, pl.ds(i * 16, 16)] = cid_v[pl.ds(i * 16, 16)]
            idxs_v[1, pl.ds(i * 16, 16)] = cid_v[pl.ds(128 + i * 16, 16)]
            return 0
        lax.fori_loop(0, 8, cpix, 0)

        cp0 = pltpu.async_copy(sim_ref.at[idxs_v.at[0]],
                               rows_v.at[pl.ds(0, 128)], sem)
        cp1 = pltpu.async_copy(sim_ref.at[idxs_v.at[1]],
                               rows_v.at[pl.ds(128, 128)], sem)

        negv = jnp.full((16,), _NEG, jnp.float32)
        def clr(i, _):
            cand_v[pl.ds(i * 16, 16)] = negv
            return 0
        lax.fori_loop(0, (_ECAP + 32) // 16, clr, 0)

        cp0.wait()
        cp1.wait()

        qrow_v = jnp.full((16,), qrow, jnp.int32)
        nch_v = nch_vec

        def fscan(g, cnt_vec):
            cid16 = cid_v[pl.ds(g * 16, 16)]
            for j2 in range(16):
                cvec = _splat(cid16, j2)
                kbase = (cvec - qrow_v) * _CH
                valid = jnp.full((16,), g * 16 + j2, jnp.int32) < nch_v
                nv = _CH // 16
                datas, msks, intras, pcs = [], [], [], []
                for v in range(nv):
                    data = rows_v[g * 16 + j2, pl.ds(v * 16, 16)]
                    msk = jnp.logical_and(data > tq, valid)
                    datas.append(data)
                    msks.append(msk)
                    intras.append(plsc.cumsum(jnp.where(msk, 1, 0)))
                    pcs.append(plsc.all_reduce_population_count(msk))
                bases = [cnt_vec]
                for v in range(nv):
                    bases.append(bases[v] + pcs[v])
                for v in range(nv):
                    pos = bases[v] + intras[v] - 1
                    kidx = kbase + (v * 16) + iota16
                    plsc.store_scatter(cand_v, [pos], datas[v], mask=msks[v])
                    plsc.store_scatter(candi_v, [pos], kidx, mask=msks[v])
                cnt_vec = jnp.minimum(bases[nv], _ECAP)
            return cnt_vec
        ngroups = (nch + 15) // 16
        lax.fori_loop(0, ngroups, fscan, zero16)

        for r in range(_ECAP // 16):
            kk = cand_v[pl.ds(r * 16, 16)]
            vv = candi_v[pl.ds(r * 16, 16)]
            sk, sv = plsc.sort_key_val(kk, vv, descending=(r % 2 == 0))
            cand_v[pl.ds(r * 16, 16)] = sk
            candi_v[pl.ds(r * 16, 16)] = sv

        nreg = _ECAP // 16
        for kv in (2, 4, 8, 16, 32):
            jv = kv // 2
            while jv >= 1:
                for r in range(nreg):
                    if r & jv:
                        continue
                    asc = (r & kv) != 0
                    ak = cand_v[pl.ds(r * 16, 16)]
                    bk = cand_v[pl.ds((r + jv) * 16, 16)]
                    av = candi_v[pl.ds(r * 16, 16)]
                    bv = candi_v[pl.ds((r + jv) * 16, 16)]
                    swap = (ak > bk) if asc else (ak < bk)
                    cand_v[pl.ds(r * 16, 16)] = jnp.where(swap, bk, ak)
                    cand_v[pl.ds((r + jv) * 16, 16)] = jnp.where(swap, ak, bk)
                    candi_v[pl.ds(r * 16, 16)] = jnp.where(swap, bv, av)
                    candi_v[pl.ds((r + jv) * 16, 16)] = jnp.where(swap, av, bv)
                jv //= 2
            for r in range(nreg):
                kk = cand_v[pl.ds(r * 16, 16)]
                vv = candi_v[pl.ds(r * 16, 16)]
                sk, sv = plsc.sort_key_val(kk, vv,
                                           descending=(r & kv) == 0)
                cand_v[pl.ds(r * 16, 16)] = sk
                candi_v[pl.ds(r * 16, 16)] = sv

        perm = (iota16 ^ 1).reshape(16, 1)
        evenlane = (iota16 & 1) == 0
        for off in (0, 1, 0):
            for r in range(13):
                base = r * 16 + off
                vv = cand_v[pl.ds(base, 16)]
                ii = candi_v[pl.ds(base, 16)]
                pv = _permute(vv, perm)
                pi = _permute(ii, perm)
                eq = vv == pv
                mn = jnp.minimum(ii, pi)
                mx = jnp.maximum(ii, pi)
                candi_v[pl.ds(base, 16)] = jnp.where(
                    eq, jnp.where(evenlane, mn, mx), ii)

        pltpu.sync_copy(cand_v.at[pl.ds(0, 256)], val_out.at[q])
        pltpu.sync_copy(candi_v.at[pl.ds(0, 256)], idx_out.at[q])
        return 0

    lax.fori_loop(0, _QPW, per_query, 0)


_sc_select = functools.partial(
    pl.kernel,
    out_type=[jax.ShapeDtypeStruct((_Q, 256), jnp.float32),
              jax.ShapeDtypeStruct((_Q, 256), jnp.int32)],
    mesh=plsc.VectorSubcoreMesh(core_axis_name="c", subcore_axis_name="s"),
    compiler_params=pltpu.CompilerParams(needs_layout_passes=False),
    scratch_types=[
        pltpu.VMEM((_QPW, _NC), jnp.float32),
        pltpu.VMEM((_QPW, 16), jnp.float32),
        pltpu.VMEM((_NCAP + 16,), jnp.int32),
        pltpu.VMEM((2, 128), jnp.int32),
        pltpu.VMEM((_NCAP, _CH), jnp.float32),
        pltpu.VMEM((_ECAP + 160,), jnp.float32),
        pltpu.VMEM((_ECAP + 160,), jnp.int32),
        pltpu.SemaphoreType.DMA,
    ],
)(_sc_body)


def kernel(queries, keys):
    qn = queries / jnp.sqrt(jnp.sum(queries ** 2, axis=-1, keepdims=True) + 1e-8)
    kn = keys / jnp.sqrt(jnp.sum(keys ** 2, axis=-1, keepdims=True) + 1e-8)

    sim, m = pl.pallas_call(
        _sim_body,
        grid=(_KP // _TK,),
        in_specs=[
            pl.BlockSpec((_Q, _D), lambda i: (0, 0)),
            pl.BlockSpec((_TK, _D), lambda i: (i, 0)),
        ],
        out_specs=[
            pl.BlockSpec((_Q, _TK // _CH, _CH), lambda i: (0, i, 0)),
            pl.BlockSpec((1, _Q, _TK // _CH), lambda i: (i, 0, 0)),
        ],
        out_shape=[
            jax.ShapeDtypeStruct((_Q, _NC, _CH), jnp.float32),
            jax.ShapeDtypeStruct((_KP // _TK, _Q, _TK // _CH), jnp.float32),
        ],
    )(qn, kn)
    m = m.transpose(1, 0, 2).reshape(_Q, _NC)

    t = pl.pallas_call(
        _thresh_body,
        out_shape=jax.ShapeDtypeStruct((_Q, 16), jnp.float32),
    )(m)

    sim2 = sim.reshape(_Q * _NC, _CH)
    val, idx = _sc_select(sim2, m, t)

    score = val[:, :_TOPK]
    end = idx[:, :_TOPK]
    matched = score > _THRESH
    matched_vocab = jnp.where(matched, end, -1)
    value = jnp.where(matched, score, 0.0)
    return score, end, matched, matched_vocab, value


# speedup vs baseline: 4.1491x; 1.0634x over previous
import functools

import jax
import jax.numpy as jnp
from jax import lax
from jax.experimental import pallas as pl
from jax.experimental.pallas import tpu as pltpu
from jax.experimental.pallas import tpu_sc as plsc

_Q, _K, _D = 1024, 100000, 128
_TOPK = 200
_THRESH = 0.05
_TK = 2048
_KP = 100352
_CH = 128
_NC = _KP // _CH
_NW = 32
_QPW = _Q // _NW
_NCAP = 256
_ECAP = 512
_NEG = -3.0


def _sim_body(q_ref, k_ref, sim_ref, m_ref):
    i = pl.program_id(0)
    qn = q_ref[...]
    kn = k_ref[...]
    sim = lax.dot_general(qn, kn, (((1,), (1,)), ((), ())),
                          preferred_element_type=jnp.float32)
    col = i * _TK + lax.broadcasted_iota(jnp.int32, (_Q, _TK), 1)
    sim = jnp.where(col < _K, sim, _NEG)
    sim3 = sim.reshape(_Q, _TK // _CH, _CH)
    sim_ref[...] = sim3
    m_ref[0] = jnp.max(sim3, axis=-1)


def _thresh_body(m_ref, t_ref):
    m = m_ref[...]

    def it(_, carry):
        lo, hi = carry
        mid = 0.5 * (lo + hi)
        cnt = jnp.sum((m > mid).astype(jnp.int32), axis=-1, keepdims=True)
        ok = cnt >= _TOPK
        return jnp.where(ok, mid, lo), jnp.where(ok, hi, mid)

    lo0 = jnp.full((_Q, 1), -1.1, jnp.float32)
    hi0 = jnp.full((_Q, 1), 1.1, jnp.float32)
    lo, _ = lax.fori_loop(0, 30, it, (lo0, hi0))
    t_ref[...] = jnp.broadcast_to(lo, (_Q, 16))


def _permute(vec16, perm16):
    dn = lax.GatherDimensionNumbers(
        offset_dims=(), collapsed_slice_dims=(0,), start_index_map=(0,))
    return lax.gather(vec16, perm16, dn, (1,),
                      mode=lax.GatherScatterMode.PROMISE_IN_BOUNDS)


def _splat(vec16, j):
    idx = jnp.full((16, 1), j, jnp.int32)
    dn = lax.GatherDimensionNumbers(
        offset_dims=(), collapsed_slice_dims=(0,), start_index_map=(0,))
    return lax.gather(vec16, idx, dn, (1,),
                      mode=lax.GatherScatterMode.PROMISE_IN_BOUNDS)


def _sc_body(sim_ref, m_ref, t_ref, val_out, idx_out,
             m_all_v, t_all_v, cid0_v, cid1_v, idxs0_v, idxs1_v,
             rows0_v, rows1_v, cand_v, candi_v, sem0, sem1):
    cid_core = lax.axis_index("c")
    sid = lax.axis_index("s")
    wid = sid * 2 + cid_core
    qbase = wid * _QPW
    iota16 = lax.iota(jnp.int32, 16)
    zero16 = jnp.zeros((16,), jnp.int32)
    nv = _CH // 16

    pltpu.sync_copy(m_ref.at[pl.ds(qbase, _QPW)], m_all_v)
    pltpu.sync_copy(t_ref.at[pl.ds(qbase, _QPW)], t_all_v)

    def initc(i, _):
        cid0_v[pl.ds(i * 16, 16)] = i * 16 + iota16
        cid1_v[pl.ds(i * 16, 16)] = i * 16 + iota16
        return 0
    lax.fori_loop(0, (_NCAP + 16) // 16, initc, 0)

    def prep(jp, cid_v, idxs_v, rows_v, sem):
        q = qbase + jp
        tq = t_all_v[jp, pl.ds(0, 16)]
        qrow = q * _NC

        def mscan(g, cnt_vec):
            m16 = m_all_v[jp, pl.ds(g * 16, 16)]
            msk = m16 > tq
            mi = jnp.where(msk, 1, 0)
            pos = cnt_vec + plsc.cumsum(mi) - 1
            ids = qrow + g * 16 + iota16
            plsc.store_scatter(cid_v, [pos], ids, mask=msk)
            pc = plsc.all_reduce_population_count(msk)
            return jnp.minimum(cnt_vec + pc, _NCAP)
        nch_vec = lax.fori_loop(0, _NC // 16, mscan, zero16)

        def cpix(i, _):
            idxs_v[0, pl.ds(i * 16, 16)] = cid_v[pl.ds(i * 16, 16)]
            idxs_v[1, pl.ds(i * 16, 16)] = cid_v[pl.ds(128 + i * 16, 16)]
            return 0
        lax.fori_loop(0, 8, cpix, 0)

        pltpu.async_copy(sim_ref.at[idxs_v.at[0]],
                         rows_v.at[pl.ds(0, 128)], sem)
        pltpu.async_copy(sim_ref.at[idxs_v.at[1]],
                         rows_v.at[pl.ds(128, 128)], sem)
        return nch_vec

    def wait_rows(idxs_v, rows_v, sem):
        pltpu.make_async_copy(sim_ref.at[idxs_v.at[0]],
                              rows_v.at[pl.ds(0, 128)], sem).wait()
        pltpu.make_async_copy(sim_ref.at[idxs_v.at[1]],
                              rows_v.at[pl.ds(128, 128)], sem).wait()

    def process(j, cid_v, idxs_v, rows_v, sem, nch_vec):
        q = qbase + j
        tq = t_all_v[j, pl.ds(0, 16)]
        qrow = q * _NC

        negv = jnp.full((16,), _NEG, jnp.float32)
        def clr(i, _):
            cand_v[pl.ds(i * 16, 16)] = negv
            return 0
        lax.fori_loop(0, (_ECAP + 32) // 16, clr, 0)

        wait_rows(idxs_v, rows_v, sem)

        qrow_v = jnp.full((16,), qrow, jnp.int32)
        nch = jnp.max(nch_vec)

        def fscan(g, cnt_vec):
            cid16 = cid_v[pl.ds(g * 16, 16)]
            for j2 in range(16):
                cvec = _splat(cid16, j2)
                kbase = (cvec - qrow_v) * _CH
                valid = jnp.full((16,), g * 16 + j2, jnp.int32) < nch_vec
                datas, msks, intras, pcs = [], [], [], []
                for v in range(nv):
                    data = rows_v[g * 16 + j2, pl.ds(v * 16, 16)]
                    msk = jnp.logical_and(data > tq, valid)
                    datas.append(data)
                    msks.append(msk)
                    intras.append(plsc.cumsum(jnp.where(msk, 1, 0)))
                    pcs.append(plsc.all_reduce_population_count(msk))
                bases = [cnt_vec]
                for v in range(nv):
                    bases.append(bases[v] + pcs[v])
                for v in range(nv):
                    pos = bases[v] + intras[v] - 1
                    kidx = kbase + (v * 16) + iota16
                    plsc.store_scatter(cand_v, [pos], datas[v], mask=msks[v])
                    plsc.store_scatter(candi_v, [pos], kidx, mask=msks[v])
                cnt_vec = jnp.minimum(bases[nv], _ECAP)
            return cnt_vec
        ngroups = (nch + 15) // 16
        lax.fori_loop(0, ngroups, fscan, zero16)

        for r in range(_ECAP // 16):
            kk = cand_v[pl.ds(r * 16, 16)]
            vv = candi_v[pl.ds(r * 16, 16)]
            sk, sv = plsc.sort_key_val(kk, vv, descending=(r % 2 == 0))
            cand_v[pl.ds(r * 16, 16)] = sk
            candi_v[pl.ds(r * 16, 16)] = sv

        nreg = _ECAP // 16
        for kv in (2, 4, 8, 16, 32):
            jv = kv // 2
            while jv >= 1:
                for r in range(nreg):
                    if r & jv:
                        continue
                    asc = (r & kv) != 0
                    ak = cand_v[pl.ds(r * 16, 16)]
                    bk = cand_v[pl.ds((r + jv) * 16, 16)]
                    av = candi_v[pl.ds(r * 16, 16)]
                    bv = candi_v[pl.ds((r + jv) * 16, 16)]
                    swap = (ak > bk) if asc else (ak < bk)
                    cand_v[pl.ds(r * 16, 16)] = jnp.where(swap, bk, ak)
                    cand_v[pl.ds((r + jv) * 16, 16)] = jnp.where(swap, ak, bk)
                    candi_v[pl.ds(r * 16, 16)] = jnp.where(swap, bv, av)
                    candi_v[pl.ds((r + jv) * 16, 16)] = jnp.where(swap, av, bv)
                jv //= 2
            for r in range(nreg):
                kk = cand_v[pl.ds(r * 16, 16)]
                vv = candi_v[pl.ds(r * 16, 16)]
                sk, sv = plsc.sort_key_val(kk, vv,
                                           descending=(r & kv) == 0)
                cand_v[pl.ds(r * 16, 16)] = sk
                candi_v[pl.ds(r * 16, 16)] = sv

        perm = (iota16 ^ 1).reshape(16, 1)
        evenlane = (iota16 & 1) == 0
        for off in (0, 1, 0):
            for r in range(13):
                base = r * 16 + off
                vv = cand_v[pl.ds(base, 16)]
                ii = candi_v[pl.ds(base, 16)]
                pv = _permute(vv, perm)
                pi = _permute(ii, perm)
                eq = vv == pv
                mn = jnp.minimum(ii, pi)
                mx = jnp.maximum(ii, pi)
                candi_v[pl.ds(base, 16)] = jnp.where(
                    eq, jnp.where(evenlane, mn, mx), ii)

        pltpu.sync_copy(cand_v.at[pl.ds(0, 256)], val_out.at[q])
        pltpu.sync_copy(candi_v.at[pl.ds(0, 256)], idx_out.at[q])

    nch_first = prep(0, cid0_v, idxs0_v, rows0_v, sem0)

    def pair(i, nchA):
        j0 = 2 * i
        nchB = prep(jnp.minimum(j0 + 1, _QPW - 1), cid1_v, idxs1_v,
                    rows1_v, sem1)
        process(j0, cid0_v, idxs0_v, rows0_v, sem0, nchA)
        nchA2 = prep(jnp.minimum(j0 + 2, _QPW - 1), cid0_v, idxs0_v,
                     rows0_v, sem0)
        process(j0 + 1, cid1_v, idxs1_v, rows1_v, sem1, nchB)
        return nchA2
    lax.fori_loop(0, _QPW // 2, pair, nch_first)
    wait_rows(idxs0_v, rows0_v, sem0)


_sc_select = functools.partial(
    pl.kernel,
    out_type=[jax.ShapeDtypeStruct((_Q, 256), jnp.float32),
              jax.ShapeDtypeStruct((_Q, 256), jnp.int32)],
    mesh=plsc.VectorSubcoreMesh(core_axis_name="c", subcore_axis_name="s"),
    compiler_params=pltpu.CompilerParams(needs_layout_passes=False),
    scratch_types=[
        pltpu.VMEM((_QPW, _NC), jnp.float32),
        pltpu.VMEM((_QPW, 16), jnp.float32),
        pltpu.VMEM((_NCAP + 16,), jnp.int32),
        pltpu.VMEM((_NCAP + 16,), jnp.int32),
        pltpu.VMEM((2, 128), jnp.int32),
        pltpu.VMEM((2, 128), jnp.int32),
        pltpu.VMEM((_NCAP, _CH), jnp.float32),
        pltpu.VMEM((_NCAP, _CH), jnp.float32),
        pltpu.VMEM((_ECAP + 160,), jnp.float32),
        pltpu.VMEM((_ECAP + 160,), jnp.int32),
        pltpu.SemaphoreType.DMA,
        pltpu.SemaphoreType.DMA,
    ],
)(_sc_body)


def kernel(queries, keys):
    qn = queries / jnp.sqrt(jnp.sum(queries ** 2, axis=-1, keepdims=True) + 1e-8)
    kn = keys / jnp.sqrt(jnp.sum(keys ** 2, axis=-1, keepdims=True) + 1e-8)

    sim, m = pl.pallas_call(
        _sim_body,
        grid=(_KP // _TK,),
        in_specs=[
            pl.BlockSpec((_Q, _D), lambda i: (0, 0)),
            pl.BlockSpec((_TK, _D), lambda i: (i, 0)),
        ],
        out_specs=[
            pl.BlockSpec((_Q, _TK // _CH, _CH), lambda i: (0, i, 0)),
            pl.BlockSpec((1, _Q, _TK // _CH), lambda i: (i, 0, 0)),
        ],
        out_shape=[
            jax.ShapeDtypeStruct((_Q, _NC, _CH), jnp.float32),
            jax.ShapeDtypeStruct((_KP // _TK, _Q, _TK // _CH), jnp.float32),
        ],
    )(qn, kn)
    m = m.transpose(1, 0, 2).reshape(_Q, _NC)

    t = pl.pallas_call(
        _thresh_body,
        out_shape=jax.ShapeDtypeStruct((_Q, 16), jnp.float32),
    )(m)

    sim2 = sim.reshape(_Q * _NC, _CH)
    val, idx = _sc_select(sim2, m, t)

    score = val[:, :_TOPK]
    end = idx[:, :_TOPK]
    matched = score > _THRESH
    matched_vocab = jnp.where(matched, end, -1)
    value = jnp.where(matched, score, 0.0)
    return score, end, matched, matched_vocab, value
